# baseline (speedup 1.0000x reference)
import functools

import jax
import jax.numpy as jnp
from jax import lax
from jax.experimental import pallas as pl
from jax.experimental.pallas import tpu as pltpu

_CHUNK = 2000


def _softmax_t(x_hbm, o_hbm, xbuf, stat, in_sem, out_sem, *, n, b):
    nch = n // _CHUNK

    def in_copy(c):
        sl = pl.ds(c * _CHUNK, _CHUNK)
        return pltpu.make_async_copy(x_hbm.at[sl], xbuf.at[sl], in_sem.at[c])

    def out_copy(c):
        sl = pl.ds(c * _CHUNK, _CHUNK)
        return pltpu.make_async_copy(xbuf.at[sl], o_hbm.at[sl], out_sem.at[c])

    for c in range(nch):
        in_copy(c).start()

    def exp_body(c, m):
        in_copy(c).wait()
        sl = pl.ds(c * _CHUNK, _CHUNK)
        x = xbuf[sl, :]
        cm = jnp.max(x, axis=0, keepdims=True)
        e = jnp.exp(x - cm)
        xbuf[sl, :] = e
        cs = jnp.sum(e, axis=0, keepdims=True)
        stat[pl.ds(8 * c, 2), :] = jnp.concatenate([cm, cs], axis=0)
        return jnp.maximum(m, cm)

    m = lax.fori_loop(
        0, nch, exp_body, jnp.full((1, b), -jnp.inf, jnp.float32)
    )

    def sum_body(c, s):
        st = stat[pl.ds(8 * c, 2), :]
        return s + st[1:2, :] * jnp.exp(st[0:1, :] - m)

    s = lax.fori_loop(0, nch, sum_body, jnp.zeros((1, b), jnp.float32))
    inv = 1.0 / s

    def scale_body(c, carry):
        sl = pl.ds(c * _CHUNK, _CHUNK)
        f = jnp.exp(stat[pl.ds(8 * c, 1), :] - m) * inv
        xbuf[sl, :] = xbuf[sl, :] * f
        out_copy(c).start()
        return carry

    lax.fori_loop(0, nch, scale_body, 0)

    def drain_body(c, carry):
        out_copy(c).wait()
        return carry

    lax.fori_loop(0, nch, drain_body, 0)


def kernel(logits):
    b, n = logits.shape
    xt = logits.T
    nch = n // _CHUNK
    out_t = pl.pallas_call(
        functools.partial(_softmax_t, n=n, b=b),
        in_specs=[pl.BlockSpec(memory_space=pl.ANY)],
        out_specs=pl.BlockSpec(memory_space=pl.ANY),
        out_shape=jax.ShapeDtypeStruct((n, b), jnp.float32),
        scratch_shapes=[
            pltpu.VMEM((n, b), jnp.float32),
            pltpu.VMEM((8 * nch, b), jnp.float32),
            pltpu.SemaphoreType.DMA((nch,)),
            pltpu.SemaphoreType.DMA((nch,)),
        ],
    )(xt)
    return out_t.T


# speedup vs baseline: 1.0118x; 1.0118x over previous
import functools

import jax
import jax.numpy as jnp
from jax import lax
from jax.experimental import pallas as pl
from jax.experimental.pallas import tpu as pltpu

_CHUNK = 10000


def _softmax_t(x_hbm, o_hbm, xbuf, stat, in_sem, out_sem, *, n, b):
    nch = n // _CHUNK

    def in_copy(c):
        sl = pl.ds(c * _CHUNK, _CHUNK)
        return pltpu.make_async_copy(x_hbm.at[sl], xbuf.at[sl], in_sem.at[c])

    def out_copy(c):
        sl = pl.ds(c * _CHUNK, _CHUNK)
        return pltpu.make_async_copy(xbuf.at[sl], o_hbm.at[sl], out_sem.at[c])

    for c in range(nch):
        in_copy(c).start()

    def exp_body(c, m):
        in_copy(c).wait()
        sl = pl.ds(c * _CHUNK, _CHUNK)
        x = xbuf[sl, :]
        cm = jnp.max(x, axis=0, keepdims=True)
        e = jnp.exp(x - cm)
        xbuf[sl, :] = e
        cs = jnp.sum(e, axis=0, keepdims=True)
        stat[pl.ds(8 * c, 2), :] = jnp.concatenate([cm, cs], axis=0)
        return jnp.maximum(m, cm)

    m = lax.fori_loop(
        0, nch, exp_body, jnp.full((1, b), -jnp.inf, jnp.float32)
    )

    def sum_body(c, s):
        st = stat[pl.ds(8 * c, 2), :]
        return s + st[1:2, :] * jnp.exp(st[0:1, :] - m)

    s = lax.fori_loop(0, nch, sum_body, jnp.zeros((1, b), jnp.float32))
    inv = 1.0 / s

    def scale_body(c, carry):
        sl = pl.ds(c * _CHUNK, _CHUNK)
        f = jnp.exp(stat[pl.ds(8 * c, 1), :] - m) * inv
        xbuf[sl, :] = xbuf[sl, :] * f
        out_copy(c).start()
        return carry

    lax.fori_loop(0, nch, scale_body, 0)

    def drain_body(c, carry):
        out_copy(c).wait()
        return carry

    lax.fori_loop(0, nch, drain_body, 0)


def kernel(logits):
    b, n = logits.shape
    xt = logits.T
    nch = n // _CHUNK
    out_t = pl.pallas_call(
        functools.partial(_softmax_t, n=n, b=b),
        in_specs=[pl.BlockSpec(memory_space=pl.ANY)],
        out_specs=pl.BlockSpec(memory_space=pl.ANY),
        out_shape=jax.ShapeDtypeStruct((n, b), jnp.float32),
        scratch_shapes=[
            pltpu.VMEM((n, b), jnp.float32),
            pltpu.VMEM((8 * nch, b), jnp.float32),
            pltpu.SemaphoreType.DMA((nch,)),
            pltpu.SemaphoreType.DMA((nch,)),
        ],
    )(xt)
    return out_t.T
